# Initial kernel scaffold; baseline (speedup 1.0000x reference)
import jax
import jax.numpy as jnp
from jax.experimental import pallas as pl


def kernel(x, embed):
    raise NotImplementedError("write your pallas kernel here")


# speedup vs baseline: 1.3837x; 1.3837x over previous
import functools

import jax
import jax.numpy as jnp
from jax import lax
from jax.experimental import pallas as pl
from jax.experimental.pallas import tpu as pltpu
from jax.experimental.pallas import tpu_sc as plsc

_NB = 1024
_CHUNKS = ((0, 2736), (2736, 5472), (5472, 8192))


def _rtne_bf16(v):
    u = lax.bitcast_convert_type(v, jnp.int32)
    r = (u + 0x7FFF + ((u >> 16) & 1)) & ~0xFFFF
    return lax.bitcast_convert_type(r, jnp.float32)


def _argmin_body(x_ref, e_ref, sx2_ref, se2_ref, idx_ref):
    x = x_ref[...]
    sum_x2 = sx2_ref[...][:, None]
    best_v = None
    best_i = None
    for lo, hi in _CHUNKS:
        c = hi - lo
        e = e_ref[pl.ds(lo, c), :]
        se2 = se2_ref[pl.ds(lo, c)]
        xe = lax.dot_general(x, e, (((1,), (1,)), ((), ())),
                             preferred_element_type=jnp.float32)
        dist = (sum_x2 - 2.0 * xe) + se2[None, :]
        m = jnp.min(dist, axis=1, keepdims=True)
        iota = lax.broadcasted_iota(jnp.int32, (_NB, c), 1) + lo
        a = jnp.min(jnp.where(dist == m, iota, jnp.int32(2**30)), axis=1)
        m = m[:, 0]
        if best_v is None:
            best_v, best_i = _rtne_bf16(m), a
        else:
            lt = m < best_v
            best_i = jnp.where(lt, a, best_i)
            best_v = jnp.where(lt, _rtne_bf16(m), best_v)
    idx_ref[...] = best_i


def _tc_argmin(xb, eb, sum_x2, sum_e2):
    n, d = xb.shape
    k = eb.shape[0]
    return pl.pallas_call(
        _argmin_body,
        grid=(n // _NB,),
        in_specs=[
            pl.BlockSpec((_NB, d), lambda i: (i, 0)),
            pl.BlockSpec((k, d), lambda i: (0, 0)),
            pl.BlockSpec((_NB,), lambda i: (i,)),
            pl.BlockSpec((k,), lambda i: (0,)),
        ],
        out_specs=pl.BlockSpec((_NB,), lambda i: (i,)),
        out_shape=jax.ShapeDtypeStruct((n,), jnp.int32),
        compiler_params=pltpu.CompilerParams(
            dimension_semantics=("arbitrary",),
        ),
    )(xb, eb, sum_x2, sum_e2)


def _sc_gather(table, idx):
    info = plsc.get_sparse_core_info()
    nw = info.num_cores * info.num_subcores
    b = idx.shape[0]
    d = table.shape[1]
    b_per_w = b // nw
    ch = 128
    n_ch = b_per_w // ch
    idx3 = idx.reshape(nw, n_ch, ch)
    mesh = plsc.VectorSubcoreMesh(core_axis_name="c", subcore_axis_name="s")

    @functools.partial(
        pl.kernel, mesh=mesh,
        out_type=jax.ShapeDtypeStruct((b, d), jnp.float32),
        scratch_types=[
            pltpu.VMEM((n_ch, ch), jnp.int32),
            pltpu.VMEM((ch, d), jnp.float32),
            pltpu.SemaphoreType.DMA,
        ],
    )
    def gather_kernel(table_hbm, idx_hbm, out_hbm, idx_v, rows_v, sem):
        wid = lax.axis_index("s") * info.num_cores + lax.axis_index("c")
        base = wid * b_per_w
        pltpu.sync_copy(idx_hbm.at[wid], idx_v)

        def chunk(c, carry):
            pltpu.async_copy(table_hbm.at[idx_v.at[c]], rows_v, sem).wait()
            pltpu.sync_copy(rows_v, out_hbm.at[pl.ds(base + c * ch, ch)])
            return carry

        lax.fori_loop(0, n_ch, chunk, 0)

    return gather_kernel(table, idx3)


def kernel(x, embed):
    shape = x.shape
    xf = x.reshape(-1, shape[-1]).astype(jnp.float32)
    ef = embed.astype(jnp.float32)
    sum_x2 = jnp.sum(xf**2, axis=1)
    sum_e2 = jnp.sum(ef.T**2, axis=0)
    flat_ind = _tc_argmin(xf.astype(jnp.bfloat16), ef.astype(jnp.bfloat16),
                          sum_x2, sum_e2)
    quantize = _sc_gather(ef, flat_ind).astype(x.dtype)
    return (quantize.reshape(shape), flat_ind.reshape(shape[:-1]))
